# baseline (speedup 1.0000x reference)
import functools

import jax
import jax.numpy as jnp
from jax import lax
from jax.experimental import pallas as pl
from jax.experimental.pallas import tpu as pltpu
from jax.experimental.pallas import tpu_sc as plsc

_N = 8192
_BQ = 256
_NBQ = _N // _BQ
_NC = 2
_NS = 16
_NW = _NC * _NS
_QPW = _N // _NW
_CH = 128
_L = 16


def _bf(x):
    return x.astype(jnp.bfloat16).astype(jnp.float32)


def _nn_tc_body(q_ref, k_ref, out_ref):
    qx = q_ref[:, 0:1]
    qy = q_ref[:, 1:2]
    kx = k_ref[0, 0:1, :]
    ky = k_ref[0, 1:2, :]
    qn = qx * qx + qy * qy
    kn = kx * kx + ky * ky
    cross = _bf(qx) * _bf(kx) + _bf(qy) * _bf(ky)
    d2 = qn + kn - 2.0 * cross
    m = jnp.min(d2, axis=1, keepdims=True)
    col = lax.broadcasted_iota(jnp.int32, d2.shape, 1)
    idx = jnp.min(jnp.where(d2 <= m, col, _N), axis=1)
    out_ref[0, 0, :] = idx


def _nn_indices(q_all, k_all):
    out = pl.pallas_call(
        _nn_tc_body,
        grid=(2 * _NBQ,),
        in_specs=[
            pl.BlockSpec((_BQ, 2), lambda i: (i, 0)),
            pl.BlockSpec((1, 2, _N), lambda i: (i // _NBQ, 0, 0)),
        ],
        out_specs=pl.BlockSpec((1, 1, _BQ), lambda i: (i, 0, 0)),
        out_shape=jax.ShapeDtypeStruct((2 * _NBQ, 1, _BQ), jnp.int32),
    )(q_all, k_all)
    return out.reshape(2, _N)


def _sc_partials(px, py, tx, ty, idx_gt, idx_pred):
    mesh = plsc.VectorSubcoreMesh(core_axis_name="c", subcore_axis_name="s")

    @functools.partial(
        pl.kernel,
        mesh=mesh,
        out_type=jax.ShapeDtypeStruct((_NW, 64), jnp.float32),
        scratch_types=[
            pltpu.VMEM((_CH,), jnp.int32),
            pltpu.VMEM((_CH,), jnp.float32),
            pltpu.VMEM((_CH,), jnp.float32),
            pltpu.VMEM((_CH,), jnp.float32),
            pltpu.VMEM((_CH,), jnp.float32),
            pltpu.VMEM((64,), jnp.float32),
            pltpu.SemaphoreType.DMA,
        ],
    )
    def body(px_h, py_h, tx_h, ty_h, ig_h, ip_h, out_h,
             idx_v, qx_v, qy_v, gx_v, gy_v, row_v, sem):
        wid = lax.axis_index("s") * _NC + lax.axis_index("c")
        base = wid * _QPW
        for d in range(2):
            qxh, qyh = (px_h, py_h) if d == 0 else (tx_h, ty_h)
            kxh, kyh = (tx_h, ty_h) if d == 0 else (px_h, py_h)
            ih = ig_h if d == 0 else ip_h
            accx = jnp.zeros((_L,), jnp.float32)
            accy = jnp.zeros((_L,), jnp.float32)
            for c in range(_QPW // _CH):
                off = base + c * _CH
                pltpu.sync_copy(ih.at[pl.ds(off, _CH)], idx_v)
                pltpu.sync_copy(qxh.at[pl.ds(off, _CH)], qx_v)
                pltpu.sync_copy(qyh.at[pl.ds(off, _CH)], qy_v)
                pltpu.async_copy(kxh.at[idx_v], gx_v, sem).wait()
                pltpu.async_copy(kyh.at[idx_v], gy_v, sem).wait()
                for i in range(_CH // _L):
                    s = pl.ds(i * _L, _L)
                    accx = accx + jnp.abs(qx_v[s] - gx_v[s])
                    accy = accy + jnp.abs(qy_v[s] - gy_v[s])
            row_v[pl.ds(d * 32, _L)] = accx
            row_v[pl.ds(d * 32 + _L, _L)] = accy
        pltpu.sync_copy(row_v, out_h.at[wid])

    return body(px, py, tx, ty, idx_gt, idx_pred)


def kernel(preds, targs, subcoef):
    p = preds[:, :2]
    t = targs[:, :2]
    q_all = jnp.concatenate([p, t], axis=0)
    k_all = jnp.stack([t.T, p.T])
    idx = _nn_indices(q_all, k_all)
    parts = _sc_partials(p[:, 0], p[:, 1], t[:, 0], t[:, 1], idx[0], idx[1])
    sums = jnp.sum(parts.reshape(_NW, 4, _L), axis=(0, 2))
    return sums[0] * subcoef[0] + sums[1] * subcoef[1] + sums[2] + sums[3]


# speedup vs baseline: 1.2996x; 1.2996x over previous
import functools

import jax
import jax.numpy as jnp
from jax import lax
from jax.experimental import pallas as pl
from jax.experimental.pallas import tpu as pltpu
from jax.experimental.pallas import tpu_sc as plsc

_N = 8192
_BQ = 256
_NBQ = _N // _BQ
_NC = 2
_NS = 16
_NW = _NC * _NS
_QPW = _N // _NW
_CH = 128
_L = 16


def _bf(x):
    return x.astype(jnp.bfloat16).astype(jnp.float32)


def _nn_tc_body(q_ref, k_ref, out_ref):
    qx = q_ref[:, 0:1]
    qy = q_ref[:, 1:2]
    kx = k_ref[0, 0:1, :]
    ky = k_ref[0, 1:2, :]
    kn = kx * kx + ky * ky
    kn1 = kn.astype(jnp.bfloat16)
    r1 = kn - kn1.astype(jnp.float32)
    kn2 = r1.astype(jnp.bfloat16)
    kn3 = (r1 - kn2.astype(jnp.float32)).astype(jnp.bfloat16)

    arow = lax.broadcasted_iota(jnp.int32, (qx.shape[0], 8), 1)
    a = jnp.where(arow == 0, _bf(qx),
        jnp.where(arow == 1, _bf(qy),
        jnp.where(arow < 5, 1.0, 0.0))).astype(jnp.bfloat16)

    brow = lax.broadcasted_iota(jnp.int32, (8, _N), 0)
    b = jnp.where(brow == 0, _bf(-2.0 * kx),
        jnp.where(brow == 1, _bf(-2.0 * ky),
        jnp.where(brow == 2, kn1.astype(jnp.float32),
        jnp.where(brow == 3, kn2.astype(jnp.float32),
        jnp.where(brow == 4, kn3.astype(jnp.float32), 0.0))))).astype(jnp.bfloat16)

    d2 = lax.dot_general(a, b, (((1,), (0,)), ((), ())),
                         preferred_element_type=jnp.float32)
    m = jnp.min(d2, axis=1, keepdims=True)
    col = lax.broadcasted_iota(jnp.int32, d2.shape, 1)
    idx = jnp.min(jnp.where(d2 <= m, col, _N), axis=1)
    out_ref[0, 0, :] = idx


def _nn_indices(q_all, k_all):
    out = pl.pallas_call(
        _nn_tc_body,
        grid=(2 * _NBQ,),
        in_specs=[
            pl.BlockSpec((_BQ, 2), lambda i: (i, 0)),
            pl.BlockSpec((1, 2, _N), lambda i: (i // _NBQ, 0, 0)),
        ],
        out_specs=pl.BlockSpec((1, 1, _BQ), lambda i: (i, 0, 0)),
        out_shape=jax.ShapeDtypeStruct((2 * _NBQ, 1, _BQ), jnp.int32),
    )(q_all, k_all)
    return out.reshape(2, _N)


def _sc_partials(px, py, tx, ty, idx_gt, idx_pred):
    mesh = plsc.VectorSubcoreMesh(core_axis_name="c", subcore_axis_name="s")

    @functools.partial(
        pl.kernel,
        mesh=mesh,
        out_type=jax.ShapeDtypeStruct((_NW, 64), jnp.float32),
        scratch_types=[
            pltpu.VMEM((_CH,), jnp.int32),
            pltpu.VMEM((_CH,), jnp.float32),
            pltpu.VMEM((_CH,), jnp.float32),
            pltpu.VMEM((_CH,), jnp.float32),
            pltpu.VMEM((_CH,), jnp.float32),
            pltpu.VMEM((64,), jnp.float32),
            pltpu.SemaphoreType.DMA,
        ],
    )
    def body(px_h, py_h, tx_h, ty_h, ig_h, ip_h, out_h,
             idx_v, qx_v, qy_v, gx_v, gy_v, row_v, sem):
        wid = lax.axis_index("s") * _NC + lax.axis_index("c")
        base = wid * _QPW
        for d in range(2):
            qxh, qyh = (px_h, py_h) if d == 0 else (tx_h, ty_h)
            kxh, kyh = (tx_h, ty_h) if d == 0 else (px_h, py_h)
            ih = ig_h if d == 0 else ip_h
            accx = jnp.zeros((_L,), jnp.float32)
            accy = jnp.zeros((_L,), jnp.float32)
            for c in range(_QPW // _CH):
                off = base + c * _CH
                pltpu.sync_copy(ih.at[pl.ds(off, _CH)], idx_v)
                pltpu.sync_copy(qxh.at[pl.ds(off, _CH)], qx_v)
                pltpu.sync_copy(qyh.at[pl.ds(off, _CH)], qy_v)
                pltpu.async_copy(kxh.at[idx_v], gx_v, sem).wait()
                pltpu.async_copy(kyh.at[idx_v], gy_v, sem).wait()
                for i in range(_CH // _L):
                    s = pl.ds(i * _L, _L)
                    accx = accx + jnp.abs(qx_v[s] - gx_v[s])
                    accy = accy + jnp.abs(qy_v[s] - gy_v[s])
            row_v[pl.ds(d * 32, _L)] = accx
            row_v[pl.ds(d * 32 + _L, _L)] = accy
        pltpu.sync_copy(row_v, out_h.at[wid])

    return body(px, py, tx, ty, idx_gt, idx_pred)


def kernel(preds, targs, subcoef):
    p = preds[:, :2]
    t = targs[:, :2]
    q_all = jnp.concatenate([p, t], axis=0)
    k_all = jnp.stack([t.T, p.T])
    idx = _nn_indices(q_all, k_all)
    parts = _sc_partials(p[:, 0], p[:, 1], t[:, 0], t[:, 1], idx[0], idx[1])
    sums = jnp.sum(parts.reshape(_NW, 4, _L), axis=(0, 2))
    return sums[0] * subcoef[0] + sums[1] * subcoef[1] + sums[2] + sums[3]
